# Initial kernel scaffold; baseline (speedup 1.0000x reference)
import jax
import jax.numpy as jnp
from jax.experimental import pallas as pl


def kernel(x, idx, label, classification_weight):
    raise NotImplementedError("write your pallas kernel here")


# speedup vs baseline: 3.0177x; 3.0177x over previous
import jax
import jax.numpy as jnp
import numpy as np
from jax.experimental import pallas as pl
from jax.experimental.pallas import tpu as pltpu

_D = 768
_C = 100
_CP = 128
_NX = 2048
_N = _C + _NX
_NT = 2176
_TB = 128
_NTILES = _NT // _TB
_K = 10
_ALPHA = 0.99
_MAXITER = 20


def _dot(a, b):
    return jnp.dot(a, b, preferred_element_type=jnp.float32)


def _lp_kernel(ap_ref, uc_ref, label_ref,
               sims_ref, pred_ref, conf_ref, acc_ref,
               ta_ref, rs_ref):
    f32 = jnp.float32

    uc = uc_ref[...]
    pmat = _dot(uc, uc.T)
    proj = _dot(ap_ref[...], pmat)
    nrm = jnp.sqrt(jnp.sum(proj * proj, axis=1, keepdims=True))
    ph = proj / (nrm + 1e-12)
    pht = ph.T

    col_ids = jax.lax.broadcasted_iota(jnp.int32, (_TB, _NT), 1)
    realc = col_ids < _N

    for i in range(_NTILES):
        sli = slice(i * _TB, (i + 1) * _TB)
        a = _dot(ph[sli, :], pht)
        row_ids = jax.lax.broadcasted_iota(jnp.int32, (_TB, _NT), 0) + i * _TB
        valid = jnp.logical_and(jnp.logical_and(row_ids < _N, realc),
                                row_ids != col_ids)
        a = jnp.where(valid, (a + 1.0) * 0.5, 0.0)

        def topk_body(_, carry):
            awork, topa = carry
            mx = jnp.max(awork, axis=1, keepdims=True)
            amx = jnp.min(jnp.where(awork == mx, col_ids, _NT),
                          axis=1, keepdims=True)
            onehot = col_ids == amx
            topa = jnp.where(onehot, mx, topa)
            awork = jnp.where(onehot, -1.0, awork)
            return awork, topa

        _, topa = jax.lax.fori_loop(0, _K, topk_body,
                                    (a, jnp.zeros((_TB, _NT), f32)))
        ta_ref[sli, :] = topa

    for i in range(_NTILES):
        sli = slice(i * _TB, (i + 1) * _TB)
        for j in range(i, _NTILES):
            slj = slice(j * _TB, (j + 1) * _TB)
            w1 = (ta_ref[sli, slj] + ta_ref[slj, sli].T) * 0.5
            ta_ref[sli, slj] = w1
            if i != j:
                ta_ref[slj, sli] = w1.T

    for i in range(_NTILES):
        sli = slice(i * _TB, (i + 1) * _TB)
        rsum = jnp.sum(ta_ref[sli, :], axis=1, keepdims=True)
        rs_ref[sli, :] = jnp.where(rsum > 0, rsum ** -0.5, 0.0)

    dcol = rs_ref[...].T
    for i in range(_NTILES):
        sli = slice(i * _TB, (i + 1) * _TB)
        dr = rs_ref[sli, :]
        ta_ref[sli, :] = dr * ta_ref[sli, :] * dcol

    nw = ta_ref[...]
    cls = jax.lax.broadcasted_iota(jnp.int32, (_CP, _NT), 0)
    node = jax.lax.broadcasted_iota(jnp.int32, (_CP, _NT), 1)
    y = jnp.where(jnp.logical_and(cls == node, cls < _C), 1.0, 0.0)

    def matvec(p):
        return p - _ALPHA * _dot(p, nw)

    xs = jnp.zeros((_CP, _NT), f32)
    r = y
    p = r
    rs = jnp.sum(r * r, axis=1, keepdims=True)

    def cg_body(_, carry):
        xs, r, p, rs = carry
        ap = matvec(p)
        alpha = rs / (jnp.sum(p * ap, axis=1, keepdims=True) + 1e-12)
        xs = xs + alpha * p
        r = r - alpha * ap
        rs_new = jnp.sum(r * r, axis=1, keepdims=True)
        beta = rs_new / (rs + 1e-12)
        p = r + beta * p
        return xs, r, p, rs_new

    xs, r, p, rs = jax.lax.fori_loop(0, _MAXITER, cg_body, (xs, r, p, rs))

    sims = xs.T
    clsn = jax.lax.broadcasted_iota(jnp.int32, (_NT, _CP), 1)
    cmask = clsn < _C
    srow = jnp.sum(jnp.where(cmask, sims, 0.0), axis=1, keepdims=True)
    norm = sims / srow
    normc = jnp.clip(norm, 1e-7, 1.0)
    ent = -normc * jnp.log(normc)
    entsum = jnp.sum(jnp.where(cmask, ent, 0.0), axis=1, keepdims=True)
    conf = 1.0 - entsum / np.log(float(_C))

    mval = jnp.max(jnp.where(cmask, sims, -1e30), axis=1, keepdims=True)
    pred = jnp.min(jnp.where(jnp.logical_and(cmask, sims == mval), clsn, _NT),
                   axis=1, keepdims=True)

    nidx = jax.lax.broadcasted_iota(jnp.int32, (_NT, 1), 0)
    is_x = jnp.logical_and(nidx >= _C, nidx < _N)
    match = jnp.logical_and(is_x, pred == label_ref[...])
    acc_ref[...] = jnp.sum(match.astype(f32), axis=0,
                           keepdims=True) * (1.0 / _NX)

    sims_ref[...] = sims
    pred_ref[...] = pred
    conf_ref[...] = conf


@jax.jit
def kernel(x, idx, label, classification_weight):
    del idx
    w = classification_weight.astype(jnp.float32)
    u, _, _ = jnp.linalg.svd(w, full_matrices=False)
    ucp = jnp.zeros((_D, _CP), jnp.float32).at[:, :63].set(u[:, 1:64])
    app = jnp.zeros((_NT, _D), jnp.float32)
    app = app.at[:_N, :].set(jnp.concatenate([w.T, x.astype(jnp.float32)], 0))
    labelp = jnp.full((_NT, 1), -1, jnp.int32).at[_C:_N, 0].set(label)

    sims_p, pred_p, conf_p, acc_p = pl.pallas_call(
        _lp_kernel,
        out_shape=[
            jax.ShapeDtypeStruct((_NT, _CP), jnp.float32),
            jax.ShapeDtypeStruct((_NT, 1), jnp.int32),
            jax.ShapeDtypeStruct((_NT, 1), jnp.float32),
            jax.ShapeDtypeStruct((1, 1), jnp.float32),
        ],
        scratch_shapes=[
            pltpu.VMEM((_NT, _NT), jnp.float32),
            pltpu.VMEM((_NT, 1), jnp.float32),
        ],
    )(app, ucp, labelp)

    out_sims = sims_p[:_N, :_C]
    pred_ex = pred_p[_C:_N, 0]
    conf_ex = conf_p[_C:_N, 0]
    return (acc_p[0, 0], out_sims, pred_ex, conf_ex)
